# baseline (speedup 1.0000x reference)
import jax
import jax.numpy as jnp
from jax import lax
from jax.experimental import pallas as pl
from jax.experimental.pallas import tpu as pltpu
from jax.experimental.pallas import tpu_sc as plsc

_NC, _NS = 2, 16
_NW = _NC * _NS
_LANES = 16
_CW = 8 * 1024
_NBUF = 7


def _sc_body(total_words):
    n_chunks_total = total_words // (_NW * _CW)

    def body(x_hbm, p_hbm, o_hbm, *rest):
        bufx = rest[:_NBUF]
        bufp = rest[_NBUF:2 * _NBUF]
        ldsem, stsem = rest[2 * _NBUF], rest[2 * _NBUF + 1]
        wid = lax.axis_index("s") * _NC + lax.axis_index("c")
        base = wid * (total_words // _NW)

        def load_descs(j):
            b = j % _NBUF
            off = base + j * _CW
            return (
                pltpu.make_async_copy(
                    x_hbm.at[pl.ds(off, _CW)], bufx[b], ldsem.at[b]),
                pltpu.make_async_copy(
                    p_hbm.at[pl.ds(off, _CW)], bufp[b], ldsem.at[b]),
            )

        def store_desc(j):
            b = j % _NBUF
            off = base + j * _CW
            return pltpu.make_async_copy(
                bufx[b], o_hbm.at[pl.ds(off, _CW)], stsem.at[b])

        for j in range(min(_NBUF - 1, n_chunks_total)):
            for d in load_descs(j):
                d.start()

        stores_waited = set()
        for j in range(n_chunks_total):
            b = j % _NBUF
            for d in load_descs(j):
                d.wait()

            @plsc.parallel_loop(0, _CW, step=_LANES, unroll=16)
            def _(i):
                plsc.addupdate(bufx[b].at[pl.ds(i, _LANES)],
                               bufp[b][pl.ds(i, _LANES)])

            nxt = j + _NBUF - 1
            if nxt < n_chunks_total:
                if j >= 1:
                    store_desc(j - 1).wait()
                    stores_waited.add(j - 1)
                for d in load_descs(nxt):
                    d.start()
            store_desc(j).start()

        for j in range(n_chunks_total):
            if j not in stores_waited:
                store_desc(j).wait()

    return body


def kernel(x, pos_emb):
    S, D = x.shape
    total = S * D
    mesh = plsc.VectorSubcoreMesh(core_axis_name="c", subcore_axis_name="s")
    run = pl.kernel(
        _sc_body(total),
        out_type=jax.ShapeDtypeStruct((total,), jnp.float32),
        mesh=mesh,
        scratch_types=(
            [pltpu.VMEM((_CW,), jnp.float32) for _ in range(2 * _NBUF)]
            + [pltpu.SemaphoreType.DMA((_NBUF,)),
               pltpu.SemaphoreType.DMA((_NBUF,))]
        ),
    )
    out = run(x.reshape(total), pos_emb.reshape(total))
    return out.reshape(1, S, D)


# speedup vs baseline: 1.2735x; 1.2735x over previous
import jax
import jax.numpy as jnp
from jax import lax
from jax.experimental import pallas as pl
from jax.experimental.pallas import tpu as pltpu
from jax.experimental.pallas import tpu_sc as plsc

_NC, _NS = 2, 16
_NW = _NC * _NS
_LANES = 16
_CW = 8 * 1024
_NBUF = 7

_F_ROWS = 1536
_BS = 512


def _sc_body(total_words):
    n_chunks_total = total_words // (_NW * _CW)

    def body(x_hbm, p_hbm, o_hbm, *rest):
        bufx = rest[:_NBUF]
        bufp = rest[_NBUF:2 * _NBUF]
        ldsem, stsem = rest[2 * _NBUF], rest[2 * _NBUF + 1]
        wid = lax.axis_index("s") * _NC + lax.axis_index("c")
        base = wid * (total_words // _NW)

        def load_descs(j):
            b = j % _NBUF
            off = base + j * _CW
            return (
                pltpu.make_async_copy(
                    x_hbm.at[pl.ds(off, _CW)], bufx[b], ldsem.at[b]),
                pltpu.make_async_copy(
                    p_hbm.at[pl.ds(off, _CW)], bufp[b], ldsem.at[b]),
            )

        def store_desc(j):
            b = j % _NBUF
            off = base + j * _CW
            return pltpu.make_async_copy(
                bufx[b], o_hbm.at[pl.ds(off, _CW)], stsem.at[b])

        for j in range(min(_NBUF - 1, n_chunks_total)):
            for d in load_descs(j):
                d.start()

        stores_waited = set()
        for j in range(n_chunks_total):
            b = j % _NBUF
            for d in load_descs(j):
                d.wait()

            @plsc.parallel_loop(0, _CW, step=_LANES, unroll=16)
            def _(i):
                plsc.addupdate(bufx[b].at[pl.ds(i, _LANES)],
                               bufp[b][pl.ds(i, _LANES)])

            nxt = j + _NBUF - 1
            if nxt < n_chunks_total:
                if j >= 1:
                    store_desc(j - 1).wait()
                    stores_waited.add(j - 1)
                for d in load_descs(nxt):
                    d.start()
            store_desc(j).start()

        for j in range(n_chunks_total):
            if j not in stores_waited:
                store_desc(j).wait()

    return body


def _sc_add_prefix(x, pos_emb, f_rows):
    S, D = x.shape
    total = f_rows * D
    mesh = plsc.VectorSubcoreMesh(core_axis_name="c", subcore_axis_name="s")
    run = pl.kernel(
        _sc_body(total),
        out_type=jax.ShapeDtypeStruct((total,), jnp.float32),
        mesh=mesh,
        scratch_types=(
            [pltpu.VMEM((_CW,), jnp.float32) for _ in range(2 * _NBUF)]
            + [pltpu.SemaphoreType.DMA((_NBUF,)),
               pltpu.SemaphoreType.DMA((_NBUF,))]
        ),
    )
    return run(x.reshape(S * D), pos_emb.reshape(S * D)).reshape(f_rows, D)


def _tc_add(x_ref, p_ref, o_ref):
    o_ref[...] = x_ref[...] + p_ref[...]


def _tc_stitch(s_ref, o1_ref, o_ref):
    o_ref[...] = s_ref[...]


def kernel(x, pos_emb):
    S, D = x.shape
    f = _F_ROWS
    skip = f // _BS

    sc_out = _sc_add_prefix(x, pos_emb, f)

    o1 = pl.pallas_call(
        _tc_add,
        grid=((S - f) // _BS,),
        in_specs=[
            pl.BlockSpec((_BS, D), lambda i: (i + skip, 0)),
            pl.BlockSpec((_BS, D), lambda i: (i + skip, 0)),
        ],
        out_specs=pl.BlockSpec((_BS, D), lambda i: (i + skip, 0)),
        out_shape=jax.ShapeDtypeStruct((S, D), x.dtype),
    )(x, pos_emb)

    out = pl.pallas_call(
        _tc_stitch,
        grid=(f // _BS,),
        in_specs=[
            pl.BlockSpec((_BS, D), lambda i: (i, 0)),
            pl.BlockSpec(memory_space=pl.ANY),
        ],
        out_specs=pl.BlockSpec((_BS, D), lambda i: (i, 0)),
        out_shape=jax.ShapeDtypeStruct((S, D), x.dtype),
        input_output_aliases={1: 0},
    )(sc_out, o1)

    return out.reshape(1, S, D)


# speedup vs baseline: 2.6163x; 2.0545x over previous
import jax
import jax.numpy as jnp
from jax import lax
from jax.experimental import pallas as pl
from jax.experimental.pallas import tpu as pltpu
from jax.experimental.pallas import tpu_sc as plsc

_NC, _NS = 2, 16
_NW = _NC * _NS
_LANES = 16
_CR = 8
_NBUF = 7


def _sc_body(S_part, D):
    rows_per_w = S_part // _NW
    n_chunks = rows_per_w // _CR

    def body(x_hbm, p_hbm, o_hbm, *rest):
        bufx = rest[:_NBUF]
        bufp = rest[_NBUF:2 * _NBUF]
        ldsem, stsem = rest[2 * _NBUF], rest[2 * _NBUF + 1]
        wid = lax.axis_index("s") * _NC + lax.axis_index("c")
        base = wid * rows_per_w

        def load_descs(j):
            b = j % _NBUF
            off = base + j * _CR
            return (
                pltpu.make_async_copy(
                    x_hbm.at[pl.ds(off, _CR)], bufx[b], ldsem.at[b]),
                pltpu.make_async_copy(
                    p_hbm.at[pl.ds(off, _CR)], bufp[b], ldsem.at[b]),
            )

        def store_desc(j):
            b = j % _NBUF
            off = base + j * _CR
            return pltpu.make_async_copy(
                bufx[b], o_hbm.at[pl.ds(off, _CR)], stsem.at[b])

        for j in range(min(_NBUF - 1, n_chunks)):
            for d in load_descs(j):
                d.start()

        stores_waited = set()
        for j in range(n_chunks):
            b = j % _NBUF
            for d in load_descs(j):
                d.wait()

            @plsc.parallel_loop(0, _CR, step=1)
            def _(r):
                @plsc.parallel_loop(0, D, step=_LANES, unroll=8)
                def _(c):
                    plsc.addupdate(bufx[b].at[r, pl.ds(c, _LANES)],
                                   bufp[b][r, pl.ds(c, _LANES)])

            nxt = j + _NBUF - 1
            if nxt < n_chunks:
                if j >= 1:
                    store_desc(j - 1).wait()
                    stores_waited.add(j - 1)
                for d in load_descs(nxt):
                    d.start()
            store_desc(j).start()

        for j in range(n_chunks):
            if j not in stores_waited:
                store_desc(j).wait()

    return body


def _sc_add_rows(x, pos_emb, s_part):
    S, D = x.shape
    mesh = plsc.VectorSubcoreMesh(core_axis_name="c", subcore_axis_name="s")
    run = pl.kernel(
        _sc_body(s_part, D),
        out_type=jax.ShapeDtypeStruct((s_part, D), jnp.float32),
        mesh=mesh,
        scratch_types=(
            [pltpu.VMEM((_CR, D), jnp.float32) for _ in range(2 * _NBUF)]
            + [pltpu.SemaphoreType.DMA((_NBUF,)),
               pltpu.SemaphoreType.DMA((_NBUF,))]
        ),
    )
    return run(x, pos_emb)


def kernel(x, pos_emb):
    S, D = x.shape
    out = _sc_add_rows(x, pos_emb, S)
    return out.reshape(1, S, D)


# speedup vs baseline: 4.7090x; 1.7998x over previous
import jax
import jax.numpy as jnp
from jax.experimental import pallas as pl

def _add_block(x_ref, p_ref, o_ref):
    o_ref[...] = x_ref[...] + p_ref[...]

def kernel(x, pos_emb):
    S, D = x.shape
    BS = 1024
    out = pl.pallas_call(
        _add_block,
        grid=(S // BS,),
        in_specs=[
            pl.BlockSpec((BS, D), lambda i: (i, 0)),
            pl.BlockSpec((BS, D), lambda i: (i, 0)),
        ],
        out_specs=pl.BlockSpec((BS, D), lambda i: (i, 0)),
        out_shape=jax.ShapeDtypeStruct((S, D), x.dtype),
    )(x, pos_emb)
    return out[None]
